# baseline (speedup 1.0000x reference)
import functools

import jax
import jax.numpy as jnp
from jax.experimental import pallas as pl
from jax.experimental.pallas import tpu as pltpu

Z_COEF = 0.001
F_CHUNK = 512


def _router_kernel(x_ref, rw_ref, rb_ref, i1_ref, i2_ref, g1_ref, g2_ref,
                   z_ref):
    S, E = x_ref.shape[0], rw_ref.shape[1]
    logits = jnp.dot(x_ref[...], rw_ref[...],
                     preferred_element_type=jnp.float32) + rb_ref[...]
    iota = jax.lax.broadcasted_iota(jnp.int32, (S, E), 1).astype(jnp.float32)
    m1 = jnp.max(logits, axis=1, keepdims=True)
    i1 = jnp.min(jnp.where(logits >= m1, iota, float(E)), axis=1,
                 keepdims=True)
    lrest = jnp.where(iota == i1, -jnp.inf, logits)
    m2 = jnp.max(lrest, axis=1, keepdims=True)
    i2 = jnp.min(jnp.where(lrest >= m2, iota, float(E)), axis=1,
                 keepdims=True)
    i1_ref[...] = i1
    i2_ref[...] = i2
    g1_ref[...] = 1.0 / (1.0 + jnp.exp(m2 - m1))
    g2_ref[...] = 1.0 / (1.0 + jnp.exp(m1 - m2))
    lse = jnp.log(jnp.sum(jnp.exp(logits - m1), axis=1, keepdims=True)) + m1
    z_ref[...] = jnp.sum(lse * lse, axis=(0, 1), keepdims=True) * (Z_COEF / S)


def _dense_kernel(xbf_ref, wi_ref, wo_ref, wib_ref, wob_ref, i1_ref, i2_ref,
                  g1_ref, g2_ref, out_ref):
    e = pl.program_id(0)
    ef = e.astype(jnp.float32)
    gate = (g1_ref[...] * (i1_ref[...] == ef) +
            g2_ref[...] * (i2_ref[...] == ef))
    xb = xbf_ref[...]
    S, H = out_ref.shape
    F = wi_ref.shape[2]
    acc = jnp.zeros((S, H), jnp.float32)
    for c in range(0, F, F_CHUNK):
        h = jnp.dot(xb, wi_ref[0, :, c:c + F_CHUNK],
                    preferred_element_type=jnp.float32)
        h = jax.nn.gelu(h + wib_ref[0, :, c:c + F_CHUNK])
        acc = acc + jnp.dot(h.astype(jnp.bfloat16),
                            wo_ref[0, c:c + F_CHUNK, :],
                            preferred_element_type=jnp.float32)
    y = (acc + wob_ref[0, :, :]) * gate

    @pl.when(e == 0)
    def _():
        out_ref[...] = y

    @pl.when(e != 0)
    def _():
        out_ref[...] += y


def kernel(x, router_w, router_b, wi_w, wi_b, wo_w, wo_b):
    B, S, H = x.shape
    E, _, F = wi_w.shape
    xs = x.reshape(S, H)

    i1, i2, g1, g2, z = pl.pallas_call(
        _router_kernel,
        out_shape=[
            jax.ShapeDtypeStruct((S, 1), jnp.float32),
            jax.ShapeDtypeStruct((S, 1), jnp.float32),
            jax.ShapeDtypeStruct((S, 1), jnp.float32),
            jax.ShapeDtypeStruct((S, 1), jnp.float32),
            jax.ShapeDtypeStruct((1, 1), jnp.float32),
        ],
    )(xs, router_w, router_b.reshape(1, E))

    xbf = xs.astype(jnp.bfloat16)
    wibf = wi_w.astype(jnp.bfloat16)
    wobf = wo_w.astype(jnp.bfloat16)

    out = pl.pallas_call(
        _dense_kernel,
        grid=(E,),
        in_specs=[
            pl.BlockSpec((S, H), lambda e: (0, 0)),
            pl.BlockSpec((1, H, F), lambda e: (e, 0, 0)),
            pl.BlockSpec((1, F, H), lambda e: (e, 0, 0)),
            pl.BlockSpec((1, 1, F), lambda e: (e, 0, 0)),
            pl.BlockSpec((1, 1, H), lambda e: (e, 0, 0)),
            pl.BlockSpec((S, 1), lambda e: (0, 0)),
            pl.BlockSpec((S, 1), lambda e: (0, 0)),
            pl.BlockSpec((S, 1), lambda e: (0, 0)),
            pl.BlockSpec((S, 1), lambda e: (0, 0)),
        ],
        out_specs=pl.BlockSpec((S, H), lambda e: (0, 0)),
        out_shape=jax.ShapeDtypeStruct((S, H), jnp.float32),
        compiler_params=pltpu.CompilerParams(
            dimension_semantics=("arbitrary",)),
    )(xbf, wibf, wobf, wi_b.reshape(E, 1, F), wo_b.reshape(E, 1, H),
      i1, i2, g1, g2)

    return out.reshape(B, S, H), z[0, 0]
